# baseline (speedup 1.0000x reference)
import functools

import jax
import jax.numpy as jnp
from jax import lax
from jax.experimental import pallas as pl
from jax.experimental.pallas import tpu as pltpu
from jax.experimental.pallas import tpu_sc as plsc

B_ROWS = 4096
T_COLS = 200
D = 128
B_TOTAL = B_ROWS * T_COLS
NC = 2
NS = 16
NW = NC * NS
ROWS_PER_W = B_TOTAL // NW
CH = 128
NCH = ROWS_PER_W // CH
NBUF = 4
LA = 2
L = 16
MAIN_ROWS = 100000
POS_BITS = 15


def _gather_body(embed_hbm, new_hbm, idx_hbm, out_hbm, idx_v, fix_v, scr_v,
                 orig_v, *bufs_and_sems):
    bufs = bufs_and_sems[:NBUF]
    gsem = bufs_and_sems[NBUF:2 * NBUF]
    ssem = bufs_and_sems[2 * NBUF:3 * NBUF]
    c = lax.axis_index("c")
    s = lax.axis_index("s")
    wid = s * NC + c
    base_row = wid * ROWS_PER_W
    pltpu.sync_copy(idx_hbm.at[pl.ds(wid * NCH, NCH)], idx_v)

    def transform(g, ngrp):
        hv = jnp.zeros((L,), jnp.int32)
        for u in range(CH // L):
            v = idx_v[g, pl.ds(u * L, L)]
            orig_v[pl.ds(u * L, L)] = v
            hv = hv | jnp.where(v >= MAIN_ROWS, 1, 0)
            idx_v[g, pl.ds(u * L, L)] = jnp.minimum(v, MAIN_ROWS - 1)
        scr_v[pl.ds(0, L)] = hv
        hr = scr_v[pl.ds(0, L)]
        has = jnp.int32(0)
        for lane in range(L):
            has = has | hr[lane]

        def compact(ngrp):
            for u in range(CH // L):
                v = orig_v[pl.ds(u * L, L)]
                m = v >= MAIN_ROWS
                pos = g * CH + u * L + lax.iota(jnp.int32, L)
                packed = ((v - MAIN_ROWS) << POS_BITS) | pos
                fix_v[pl.ds(ngrp * L, L)] = jnp.where(m, packed, -1)
                gh = jnp.int32(0)
                for lane in range(L):
                    gh = gh | jnp.where(v[lane] >= MAIN_ROWS, 1, 0)
                ngrp = ngrp + gh
            return ngrp

        return lax.cond(has > 0, compact, lambda n: n, ngrp)

    def gstart(g, j):
        pltpu.async_copy(embed_hbm.at[idx_v.at[g]], bufs[j], gsem[j])

    def gwait(j):
        pltpu.make_async_copy(embed_hbm.at[idx_v.at[0]], bufs[j],
                              gsem[j]).wait()

    def sstart(g, j):
        pltpu.async_copy(bufs[j], out_hbm.at[pl.ds(base_row + g * CH, CH)],
                         ssem[j])

    def swait(j):
        pltpu.make_async_copy(bufs[j], out_hbm.at[pl.ds(0, CH)],
                              ssem[j]).wait()

    def step(g, j, nfix, do_swait, do_gahead):
        ja = (j + LA) % NBUF
        if do_gahead:
            if do_swait:
                swait(ja)
            nfix = transform(g + LA, nfix)
            gstart(g + LA, ja)
        gwait(j)
        sstart(g, j)
        return nfix

    nfix = jnp.int32(0)
    for g in range(LA):
        nfix = transform(g, nfix)
        gstart(g, g % NBUF)
    for j in range(NBUF):
        nfix = step(j, j, nfix, do_swait=(j + LA >= NBUF), do_gahead=True)

    def body(q, nfix):
        g0 = q * NBUF
        for j in range(NBUF):
            nfix = step(g0 + j, j, nfix, do_swait=True, do_gahead=True)
        return nfix

    nfix = lax.fori_loop(1, NCH // NBUF - 1, body, nfix)

    g0 = NCH - NBUF
    for j in range(NBUF):
        nfix = step(g0 + j, j, nfix, do_swait=True, do_gahead=(j + LA < NBUF))
    for j in range(NBUF):
        swait(j)

    def fix_batch(i, carry):
        v16 = fix_v[pl.ds(i * L, L)]
        n = jnp.int32(0)
        for lane in range(L):
            n = n + jnp.where(v16[lane] >= 0, 1, 0)

        def drain(r, carry):
            pltpu.make_async_copy(new_hbm.at[pl.ds(0, 1)],
                                  bufs[0].at[pl.ds(0, 1)], gsem[0]).wait()
            return carry

        def sdrain(r, carry):
            pltpu.make_async_copy(bufs[0].at[pl.ds(0, 1)],
                                  out_hbm.at[pl.ds(0, 1)], ssem[0]).wait()
            return carry

        for lane in range(L):
            @pl.when(v16[lane] >= 0)
            def _():
                pltpu.async_copy(new_hbm.at[pl.ds(v16[lane] >> POS_BITS, 1)],
                                 bufs[0].at[pl.ds(lane, 1)], gsem[0])
        lax.fori_loop(0, n, drain, 0)
        for lane in range(L):
            @pl.when(v16[lane] >= 0)
            def _():
                pos = v16[lane] & ((1 << POS_BITS) - 1)
                pltpu.async_copy(bufs[0].at[pl.ds(lane, 1)],
                                 out_hbm.at[pl.ds(base_row + pos, 1)], ssem[0])
        lax.fori_loop(0, n, sdrain, 0)
        return carry

    lax.fori_loop(0, nfix, fix_batch, 0)


@jax.jit
def kernel(input, embed_weight, new_embed_weight):
    idx = input.reshape(-1).astype(jnp.int32).reshape(B_TOTAL // CH, CH)
    mesh = plsc.VectorSubcoreMesh(core_axis_name="c", subcore_axis_name="s")
    run = pl.kernel(
        _gather_body,
        out_type=jax.ShapeDtypeStruct((B_TOTAL, D), jnp.float32),
        mesh=mesh,
        scratch_types=(
            [pltpu.VMEM((NCH, CH), jnp.int32),
             pltpu.VMEM((ROWS_PER_W + L,), jnp.int32),
             pltpu.VMEM((L,), jnp.int32),
             pltpu.VMEM((CH,), jnp.int32)]
            + [pltpu.VMEM((CH, D), jnp.float32)] * NBUF
            + [pltpu.SemaphoreType.DMA] * (2 * NBUF)
        ),
    )
    out = run(embed_weight, new_embed_weight, idx)
    return out.reshape(B_ROWS, T_COLS, D)


# speedup vs baseline: 1.0049x; 1.0049x over previous
import jax
import jax.numpy as jnp
from jax import lax
from jax.experimental import pallas as pl
from jax.experimental.pallas import tpu as pltpu
from jax.experimental.pallas import tpu_sc as plsc

B_ROWS = 4096
T_COLS = 200
D = 128
B_TOTAL = B_ROWS * T_COLS
NC = 2
NS = 16
NW = NC * NS
ROWS_PER_W = B_TOTAL // NW
CH = 128
NCH = ROWS_PER_W // CH
NBUF = 4
LA = 2
L = 16
MAIN_ROWS = 100000
POS_BITS = 15


def _gather_body(embed_hbm, new_hbm, idx_hbm, out_hbm, idx_v, fix_v, scr_v,
                 orig_v, *bufs_and_sems):
    bufs = bufs_and_sems[:NBUF]
    gsem = bufs_and_sems[NBUF:2 * NBUF]
    ssem = bufs_and_sems[2 * NBUF:3 * NBUF]
    c = lax.axis_index("c")
    s = lax.axis_index("s")
    wid = s * NC + c
    base_row = wid * ROWS_PER_W
    pltpu.sync_copy(idx_hbm.at[pl.ds(wid * NCH, NCH)], idx_v)

    def transform(g, ngrp):
        hv = jnp.zeros((L,), jnp.int32)
        for u in range(CH // L):
            v = idx_v[g, pl.ds(u * L, L)]
            orig_v[pl.ds(u * L, L)] = v
            hv = hv | jnp.where(v >= MAIN_ROWS, 1, 0)
            idx_v[g, pl.ds(u * L, L)] = jnp.minimum(v, MAIN_ROWS - 1)
        scr_v[pl.ds(0, L)] = hv
        hr = scr_v[pl.ds(0, L)]
        has = jnp.int32(0)
        for lane in range(L):
            has = has | hr[lane]

        def compact(ngrp):
            for u in range(CH // L):
                v = orig_v[pl.ds(u * L, L)]
                m = v >= MAIN_ROWS
                pos = g * CH + u * L + lax.iota(jnp.int32, L)
                packed = ((v - MAIN_ROWS) << POS_BITS) | pos
                fix_v[pl.ds(ngrp * L, L)] = jnp.where(m, packed, -1)
                gh = jnp.int32(0)
                for lane in range(L):
                    gh = gh | jnp.where(v[lane] >= MAIN_ROWS, 1, 0)
                ngrp = ngrp + gh
            return ngrp

        return lax.cond(has > 0, compact, lambda n: n, ngrp)

    def gstart(g, j):
        pltpu.async_copy(embed_hbm.at[idx_v.at[g]], bufs[j], gsem[j])

    def gwait(j):
        pltpu.make_async_copy(embed_hbm.at[idx_v.at[0]], bufs[j],
                              gsem[j]).wait()

    def sstart(g, j):
        pltpu.async_copy(bufs[j], out_hbm.at[pl.ds(base_row + g * CH, CH)],
                         ssem[j])

    def swait(j):
        pltpu.make_async_copy(bufs[j], out_hbm.at[pl.ds(0, CH)],
                              ssem[j]).wait()

    def step(g, j, nfix, do_swait, do_gahead):
        ja = (j + LA) % NBUF
        if do_gahead:
            if do_swait:
                swait(ja)
            nfix = transform(g + LA, nfix)
            gstart(g + LA, ja)
        gwait(j)
        sstart(g, j)
        return nfix

    nfix = jnp.int32(0)
    for g in range(LA):
        nfix = transform(g, nfix)
        gstart(g, g % NBUF)
    for j in range(NBUF):
        nfix = step(j, j, nfix, do_swait=(j + LA >= NBUF), do_gahead=True)

    def body(q, nfix):
        g0 = q * NBUF
        for j in range(NBUF):
            nfix = step(g0 + j, j, nfix, do_swait=True, do_gahead=True)
        return nfix

    nfix = lax.fori_loop(1, NCH // NBUF - 1, body, nfix)

    g0 = NCH - NBUF
    for j in range(NBUF):
        nfix = step(g0 + j, j, nfix, do_swait=True, do_gahead=(j + LA < NBUF))
    for j in range(NBUF):
        swait(j)

    def fix_batch(i, carry):
        v16 = fix_v[pl.ds(i * L, L)]
        n = jnp.int32(0)
        for lane in range(L):
            n = n + jnp.where(v16[lane] >= 0, 1, 0)

        def drain(r, carry):
            pltpu.make_async_copy(new_hbm.at[pl.ds(0, 1)],
                                  bufs[0].at[pl.ds(0, 1)], gsem[0]).wait()
            return carry

        def sdrain(r, carry):
            pltpu.make_async_copy(bufs[0].at[pl.ds(0, 1)],
                                  out_hbm.at[pl.ds(0, 1)], ssem[0]).wait()
            return carry

        for lane in range(L):
            @pl.when(v16[lane] >= 0)
            def _():
                pltpu.async_copy(new_hbm.at[pl.ds(v16[lane] >> POS_BITS, 1)],
                                 bufs[0].at[pl.ds(lane, 1)], gsem[0])
        lax.fori_loop(0, n, drain, 0)
        for lane in range(L):
            @pl.when(v16[lane] >= 0)
            def _():
                pos = v16[lane] & ((1 << POS_BITS) - 1)
                pltpu.async_copy(bufs[0].at[pl.ds(lane, 1)],
                                 out_hbm.at[pl.ds(base_row + pos, 1)], ssem[0])
        lax.fori_loop(0, n, sdrain, 0)
        return carry

    lax.fori_loop(0, nfix, fix_batch, 0)


@jax.jit
def kernel(input, embed_weight, new_embed_weight):
    idx = input.reshape(-1).astype(jnp.int32).reshape(B_TOTAL // CH, CH)
    mesh = plsc.VectorSubcoreMesh(core_axis_name="c", subcore_axis_name="s")
    run = pl.kernel(
        _gather_body,
        out_type=jax.ShapeDtypeStruct((B_TOTAL, D), jnp.float32),
        mesh=mesh,
        scratch_types=(
            [pltpu.VMEM((NCH, CH), jnp.int32),
             pltpu.VMEM((ROWS_PER_W + L,), jnp.int32),
             pltpu.VMEM((L,), jnp.int32),
             pltpu.VMEM((CH,), jnp.int32)]
            + [pltpu.VMEM((CH, D), jnp.float32)] * NBUF
            + [pltpu.SemaphoreType.DMA] * (2 * NBUF)
        ),
    )
    out = run(embed_weight, new_embed_weight, idx)
    return out.reshape(B_ROWS, T_COLS, D)
